# baseline (speedup 1.0000x reference)
import functools

import jax
import jax.numpy as jnp
from jax import lax
from jax.experimental import pallas as pl
from jax.experimental.pallas import tpu as pltpu
from jax.experimental.pallas import tpu_sc as plsc

N = 100000
EMB = 128
NTILES = 32
ROWS_PER_TILE = 3200
LAST_TILE_GROUPS = 50
FULL_GROUPS = 200
NPAD = NTILES * ROWS_PER_TILE

OFF1 = 119
OFF2 = 119 + 60
OFF3 = 119 + 60 + 120
TOTAL_ROWS = 119 + 60 + 120 + 144


def _sc_body(xt_ref, tab_ref, out_ref, tab_v, x_v, obuf, sem0, sem1):
    wid = lax.axis_index("s") * 2 + lax.axis_index("c")
    base = wid * ROWS_PER_TILE

    pltpu.sync_copy(tab_ref, tab_v)
    pltpu.sync_copy(xt_ref.at[:, pl.ds(base, ROWS_PER_TILE)], x_v)

    ngroups = jnp.where(wid == NTILES - 1, LAST_TILE_GROUPS, FULL_GROUPS)
    sems = (sem0, sem1)

    def compute_group(g, b):
        gb = g * 16
        x0 = x_v[0, pl.ds(gb, 16)]
        x1 = x_v[1, pl.ds(gb, 16)]
        x2 = x_v[2, pl.ds(gb, 16)]
        x3 = x_v[3, pl.ds(gb, 16)]
        x4 = x_v[4, pl.ds(gb, 16)]
        x5 = x_v[5, pl.ds(gb, 16)]
        x6 = x_v[6, pl.ds(gb, 16)]
        x7 = x_v[7, pl.ds(gb, 16)]
        x8 = x_v[8, pl.ds(gb, 16)]
        v0 = x0
        v1 = x1 * 12 + x2 + OFF1
        v2 = x3 * 10 + x4 + OFF2
        v3 = ((x5 * 6 + x6) * 2 + x7) * 2 + x8 + OFF3

        for r in range(16):
            i0 = v0[r]
            i1 = v1[r]
            i2 = v2[r]
            i3 = v3[r]
            for k in range(EMB // 16):
                cs = pl.ds(k * 16, 16)
                acc = tab_v[i0, cs] + tab_v[i1, cs]
                acc = acc + tab_v[i2, cs]
                acc = acc + tab_v[i3, cs]
                obuf[b, r, cs] = acc

        pltpu.make_async_copy(
            obuf.at[b], out_ref.at[pl.ds(base + gb, 16), :], sems[b]
        ).start()

    compute_group(0, 0)
    compute_group(1, 1)

    def h_body(h, carry):
        for b in range(2):
            g = h * 2 + b
            pltpu.make_async_copy(
                obuf.at[b], out_ref.at[pl.ds(base, 16), :], sems[b]
            ).wait()
            compute_group(g, b)
        return carry

    lax.fori_loop(1, ngroups // 2, h_body, 0)
    pltpu.make_async_copy(obuf.at[0], out_ref.at[pl.ds(base, 16), :], sem0).wait()
    pltpu.make_async_copy(obuf.at[1], out_ref.at[pl.ds(base, 16), :], sem1).wait()


@jax.jit
def kernel(x, W0, W1, W2, W3, W4, W5, W6, W7, W8):
    t12 = (W1[:, None, :] + W2[None, :, :]).reshape(60, EMB)
    t34 = (W3[:, None, :] + W4[None, :, :]).reshape(120, EMB)
    t5678 = (
        W5[:, None, None, None, :]
        + W6[None, :, None, None, :]
        + W7[None, None, :, None, :]
        + W8[None, None, None, :, :]
    ).reshape(144, EMB)
    tab = jnp.concatenate([W0, t12, t34, t5678], axis=0)

    xt = jnp.pad(x, ((0, NPAD - N), (0, 0))).T

    mesh = plsc.VectorSubcoreMesh(core_axis_name="c", subcore_axis_name="s")
    run = pl.kernel(
        _sc_body,
        out_type=jax.ShapeDtypeStruct((N, EMB), jnp.float32),
        mesh=mesh,
        scratch_types=[
            pltpu.VMEM((TOTAL_ROWS, EMB), jnp.float32),
            pltpu.VMEM((9, ROWS_PER_TILE), jnp.int32),
            pltpu.VMEM((2, 16, EMB), jnp.float32),
            pltpu.SemaphoreType.DMA,
            pltpu.SemaphoreType.DMA,
        ],
    )
    return run(xt, tab)


# speedup vs baseline: 1.6688x; 1.6688x over previous
import functools

import jax
import jax.numpy as jnp
from jax import lax
from jax.experimental import pallas as pl
from jax.experimental.pallas import tpu as pltpu
from jax.experimental.pallas import tpu_sc as plsc

N = 100000
EMB = 128
NTILES = 32
ROWS_PER_TILE = 3200
LAST_TILE_GROUPS = 50
FULL_GROUPS = 200
NPAD = NTILES * ROWS_PER_TILE

OFF1 = 119
OFF2 = 119 + 60
OFF3 = 119 + 60 + 120
TOTAL_ROWS = 119 + 60 + 120 + 144


def _sc_body(xt_ref, tab_ref, out_ref, tab_v, x_v, obuf, sem):
    wid = lax.axis_index("s") * 2 + lax.axis_index("c")
    base = wid * ROWS_PER_TILE

    pltpu.sync_copy(tab_ref, tab_v)
    pltpu.sync_copy(xt_ref.at[:, pl.ds(base, ROWS_PER_TILE)], x_v)

    ngroups = jnp.where(wid == NTILES - 1, LAST_TILE_GROUPS, FULL_GROUPS)

    def compute_group(g, b):
        gb = g * 16
        x0 = x_v[0, pl.ds(gb, 16)]
        x1 = x_v[1, pl.ds(gb, 16)]
        x2 = x_v[2, pl.ds(gb, 16)]
        x3 = x_v[3, pl.ds(gb, 16)]
        x4 = x_v[4, pl.ds(gb, 16)]
        x5 = x_v[5, pl.ds(gb, 16)]
        x6 = x_v[6, pl.ds(gb, 16)]
        x7 = x_v[7, pl.ds(gb, 16)]
        x8 = x_v[8, pl.ds(gb, 16)]
        v0 = x0
        v1 = x1 * 12 + x2 + OFF1
        v2 = x3 * 10 + x4 + OFF2
        v3 = ((x5 * 6 + x6) * 2 + x7) * 2 + x8 + OFF3

        for r in range(16):
            i0 = v0[r]
            i1 = v1[r]
            i2 = v2[r]
            i3 = v3[r]
            for k in range(EMB // 16):
                cs = pl.ds(k * 16, 16)
                acc = tab_v[i0, cs] + tab_v[i1, cs]
                acc = acc + tab_v[i2, cs]
                acc = acc + tab_v[i3, cs]
                obuf[b, r, cs] = acc

        pltpu.make_async_copy(
            obuf.at[b], out_ref.at[pl.ds(base + gb, 16), :], sem.at[b]
        ).start()

    compute_group(0, 0)
    compute_group(1, 1)

    def g_body(g, carry):
        b = g % 2
        pltpu.make_async_copy(
            obuf.at[b], out_ref.at[pl.ds(base, 16), :], sem.at[b]
        ).wait()
        compute_group(g, b)
        return carry

    lax.fori_loop(2, ngroups, g_body, 0)
    pltpu.make_async_copy(obuf.at[0], out_ref.at[pl.ds(base, 16), :], sem.at[0]).wait()
    pltpu.make_async_copy(obuf.at[1], out_ref.at[pl.ds(base, 16), :], sem.at[1]).wait()


@jax.jit
def kernel(x, W0, W1, W2, W3, W4, W5, W6, W7, W8):
    t12 = (W1[:, None, :] + W2[None, :, :]).reshape(60, EMB)
    t34 = (W3[:, None, :] + W4[None, :, :]).reshape(120, EMB)
    t5678 = (
        W5[:, None, None, None, :]
        + W6[None, :, None, None, :]
        + W7[None, None, :, None, :]
        + W8[None, None, None, :, :]
    ).reshape(144, EMB)
    tab = jnp.concatenate([W0, t12, t34, t5678], axis=0)

    xt = jnp.pad(x, ((0, NPAD - N), (0, 0))).T

    mesh = plsc.VectorSubcoreMesh(core_axis_name="c", subcore_axis_name="s")
    run = pl.kernel(
        _sc_body,
        out_type=jax.ShapeDtypeStruct((N, EMB), jnp.float32),
        mesh=mesh,
        scratch_types=[
            pltpu.VMEM((TOTAL_ROWS, EMB), jnp.float32),
            pltpu.VMEM((9, ROWS_PER_TILE), jnp.int32),
            pltpu.VMEM((2, 16, EMB), jnp.float32),
            pltpu.SemaphoreType.DMA((2,)),
        ],
    )
    return run(xt, tab)
